# speedup vs baseline: 2.2756x; 2.2756x over previous
import functools

import jax
import jax.numpy as jnp
from jax import lax
from jax.experimental import pallas as pl
from jax.experimental.pallas import tpu as pltpu
from jax.experimental.pallas import tpu_sc as plsc

SEQ = 8192
BATCH = 4
D = 768
HALF = D // 2
MAX_BAR_POS = 96
MAX_BARS = 32
N_ROWS = SEQ * BATCH
N_TABLE = MAX_BAR_POS * MAX_BARS

NC = 2
NS = 16
NW = NC * NS
ROWS_PER_W = N_ROWS // NW
CHUNK = 64
NCHUNK = ROWS_PER_W // CHUNK


def _prep_body(pos_t_ref, num_t_ref, w_ref, b_ref, pi_ref, bi_ref,
               table_ref, idx_ref):
    wp = w_ref[:, :HALF]
    wn = w_ref[:, HALF:]
    p = lax.dot_general(pos_t_ref[:], wp, (((1,), (1,)), ((), ())),
                        preferred_element_type=jnp.float32) + b_ref[:]
    n = lax.dot_general(num_t_ref[:], wn, (((1,), (1,)), ((), ())),
                        preferred_element_type=jnp.float32)
    table_ref[:] = p[:, None, :] + n[None, :, :]
    idx_ref[:] = pi_ref[:] * MAX_BARS + bi_ref[:]


def _prep(pos_t, num_t, w, b2, pos_i, bar_i):
    return pl.pallas_call(
        _prep_body,
        out_shape=(
            jax.ShapeDtypeStruct((MAX_BAR_POS, MAX_BARS, D), jnp.float32),
            jax.ShapeDtypeStruct((N_ROWS // 128, 128), jnp.int32),
        ),
    )(pos_t, num_t, w, b2, pos_i, bar_i)


@functools.partial(
    pl.kernel,
    out_type=jax.ShapeDtypeStruct((N_ROWS, D), jnp.float32),
    mesh=plsc.VectorSubcoreMesh(core_axis_name="c", subcore_axis_name="s"),
    scratch_types=[
        pltpu.VMEM((2, CHUNK), jnp.int32),
        pltpu.VMEM((2, CHUNK, D), jnp.float32),
        pltpu.SemaphoreType.DMA((2,)),
        pltpu.SemaphoreType.DMA((2,)),
    ],
)
def _sc_gather(table_hbm, idx_hbm, out_hbm, idx_v, rows_v, gsem, wsem):
    wid = lax.axis_index("s") * NC + lax.axis_index("c")
    base = wid * ROWS_PER_W
    for c in range(NCHUNK):
        off = base + c * CHUNK
        buf = c % 2
        pltpu.sync_copy(idx_hbm.at[pl.ds(off, CHUNK)], idx_v.at[buf])
        g = pltpu.async_copy(table_hbm.at[idx_v.at[buf]], rows_v.at[buf],
                             gsem.at[buf])
        g.wait()
        w = pltpu.async_copy(rows_v.at[buf], out_hbm.at[pl.ds(off, CHUNK)],
                             wsem.at[buf])
        w.wait()


def kernel(bar_positions, bar_numbers, bar_position_table, bar_number_table,
           W, b):
    pos_i = bar_positions.reshape(N_ROWS // 128, 128).astype(jnp.int32)
    bar_i = bar_numbers.reshape(N_ROWS // 128, 128).astype(jnp.int32)
    table3, idx2 = _prep(bar_position_table, bar_number_table, W,
                         b.reshape(1, D), pos_i, bar_i)
    out_flat = _sc_gather(table3.reshape(N_TABLE, D), idx2.reshape(N_ROWS))
    return out_flat.reshape(SEQ, BATCH, D)


# speedup vs baseline: 2.4153x; 1.0614x over previous
import functools

import jax
import jax.numpy as jnp
from jax import lax
from jax.experimental import pallas as pl
from jax.experimental.pallas import tpu as pltpu
from jax.experimental.pallas import tpu_sc as plsc

SEQ = 8192
BATCH = 4
D = 768
HALF = D // 2
MAX_BAR_POS = 96
MAX_BARS = 32
N_ROWS = SEQ * BATCH
N_TABLE = MAX_BAR_POS * MAX_BARS

NC = 2
NS = 16
NW = NC * NS
ROWS_PER_W = N_ROWS // NW
CHUNK = 64
NCHUNK = ROWS_PER_W // CHUNK


def _prep_body(pos_t_ref, num_t_ref, w_ref, b_ref, pi_ref, bi_ref,
               table_ref, idx_ref):
    wp = w_ref[:, :HALF]
    wn = w_ref[:, HALF:]
    p = lax.dot_general(pos_t_ref[:], wp, (((1,), (1,)), ((), ())),
                        preferred_element_type=jnp.float32) + b_ref[:]
    n = lax.dot_general(num_t_ref[:], wn, (((1,), (1,)), ((), ())),
                        preferred_element_type=jnp.float32)
    table_ref[:] = p[:, None, :] + n[None, :, :]
    idx_ref[:] = pi_ref[:] * MAX_BARS + bi_ref[:]


def _prep(pos_t, num_t, w, b2, pos_i, bar_i):
    return pl.pallas_call(
        _prep_body,
        out_shape=(
            jax.ShapeDtypeStruct((MAX_BAR_POS, MAX_BARS, D), jnp.float32),
            jax.ShapeDtypeStruct((N_ROWS // 128, 128), jnp.int32),
        ),
    )(pos_t, num_t, w, b2, pos_i, bar_i)


@functools.partial(
    pl.kernel,
    out_type=jax.ShapeDtypeStruct((N_ROWS, D), jnp.float32),
    mesh=plsc.VectorSubcoreMesh(core_axis_name="c", subcore_axis_name="s"),
    scratch_types=[
        pltpu.VMEM((2, CHUNK), jnp.int32),
        pltpu.VMEM((2, CHUNK, D), jnp.float32),
        pltpu.SemaphoreType.DMA((2,)),
        pltpu.SemaphoreType.DMA((2,)),
    ],
)
def _sc_gather(table_hbm, idx_hbm, out_hbm, idx_v, rows_v, gsem, wsem):
    wid = lax.axis_index("s") * NC + lax.axis_index("c")
    base = wid * ROWS_PER_W
    pltpu.sync_copy(idx_hbm.at[pl.ds(base, CHUNK)], idx_v.at[0])
    g_cur = pltpu.async_copy(table_hbm.at[idx_v.at[0]], rows_v.at[0],
                             gsem.at[0])
    writes = [None, None]
    for c in range(NCHUNK):
        buf = c % 2
        nbuf = (c + 1) % 2
        if c + 1 < NCHUNK:
            noff = base + (c + 1) * CHUNK
            pltpu.sync_copy(idx_hbm.at[pl.ds(noff, CHUNK)], idx_v.at[nbuf])
            if writes[nbuf] is not None:
                writes[nbuf].wait()
            g_next = pltpu.async_copy(table_hbm.at[idx_v.at[nbuf]],
                                      rows_v.at[nbuf], gsem.at[nbuf])
        g_cur.wait()
        writes[buf] = pltpu.async_copy(
            rows_v.at[buf], out_hbm.at[pl.ds(base + c * CHUNK, CHUNK)],
            wsem.at[buf])
        if c + 1 < NCHUNK:
            g_cur = g_next
    for w in writes:
        if w is not None:
            w.wait()


def kernel(bar_positions, bar_numbers, bar_position_table, bar_number_table,
           W, b):
    pos_i = bar_positions.reshape(N_ROWS // 128, 128).astype(jnp.int32)
    bar_i = bar_numbers.reshape(N_ROWS // 128, 128).astype(jnp.int32)
    table3, idx2 = _prep(bar_position_table, bar_number_table, W,
                         b.reshape(1, D), pos_i, bar_i)
    out_flat = _sc_gather(table3.reshape(N_TABLE, D), idx2.reshape(N_ROWS))
    return out_flat.reshape(SEQ, BATCH, D)


# speedup vs baseline: 4.7094x; 1.9498x over previous
import functools

import jax
import jax.numpy as jnp
from jax import lax
from jax.experimental import pallas as pl
from jax.experimental.pallas import tpu as pltpu
from jax.experimental.pallas import tpu_sc as plsc

SEQ = 8192
BATCH = 4
D = 768
HALF = D // 2
MAX_BAR_POS = 96
MAX_BARS = 32
N_ROWS = SEQ * BATCH
N_TABLE = MAX_BAR_POS * MAX_BARS

NC = 2
NS = 16
NW = NC * NS
ROWS_PER_W = N_ROWS // NW
CHUNK = 64
NCHUNK = ROWS_PER_W // CHUNK


def _prep_body(pos_t_ref, num_t_ref, w_ref, b_ref, pi_ref, bi_ref,
               table_ref, idx_ref):
    wp = w_ref[:, :HALF]
    wn = w_ref[:, HALF:]
    p = lax.dot_general(pos_t_ref[:], wp, (((1,), (1,)), ((), ())),
                        preferred_element_type=jnp.float32) + b_ref[:]
    n = lax.dot_general(num_t_ref[:], wn, (((1,), (1,)), ((), ())),
                        preferred_element_type=jnp.float32)
    table_ref[:] = p[:, None, :] + n[None, :, :]
    idx_ref[:] = pi_ref[:] * MAX_BARS + bi_ref[:]


def _prep(pos_t, num_t, w, b2, pos_i, bar_i):
    return pl.pallas_call(
        _prep_body,
        out_shape=(
            jax.ShapeDtypeStruct((MAX_BAR_POS, MAX_BARS, D), jnp.float32),
            jax.ShapeDtypeStruct((N_ROWS // 128, 128), jnp.int32),
        ),
    )(pos_t, num_t, w, b2, pos_i, bar_i)


@functools.partial(
    pl.kernel,
    out_type=jax.ShapeDtypeStruct((SEQ, BATCH, D), jnp.float32),
    mesh=plsc.VectorSubcoreMesh(core_axis_name="c", subcore_axis_name="s"),
    scratch_types=[
        pltpu.VMEM((2, CHUNK), jnp.int32),
        pltpu.VMEM((2, CHUNK, D), jnp.float32),
        pltpu.SemaphoreType.DMA((2,)),
        pltpu.SemaphoreType.DMA((2,)),
    ],
)
def _sc_gather(table_hbm, idx_hbm, out3_hbm, idx_v, rows_v, gsem, wsem):
    out_hbm = out3_hbm.reshape(N_ROWS, D)
    wid = lax.axis_index("s") * NC + lax.axis_index("c")
    base = wid * ROWS_PER_W
    pltpu.sync_copy(idx_hbm.at[pl.ds(base, CHUNK)], idx_v.at[0])
    g_cur = pltpu.async_copy(table_hbm.at[idx_v.at[0]], rows_v.at[0],
                             gsem.at[0])
    writes = [None, None]
    for c in range(NCHUNK):
        buf = c % 2
        nbuf = (c + 1) % 2
        if c + 1 < NCHUNK:
            noff = base + (c + 1) * CHUNK
            pltpu.sync_copy(idx_hbm.at[pl.ds(noff, CHUNK)], idx_v.at[nbuf])
            if writes[nbuf] is not None:
                writes[nbuf].wait()
            g_next = pltpu.async_copy(table_hbm.at[idx_v.at[nbuf]],
                                      rows_v.at[nbuf], gsem.at[nbuf])
        g_cur.wait()
        writes[buf] = pltpu.async_copy(
            rows_v.at[buf], out_hbm.at[pl.ds(base + c * CHUNK, CHUNK)],
            wsem.at[buf])
        if c + 1 < NCHUNK:
            g_cur = g_next
    for w in writes:
        if w is not None:
            w.wait()


def kernel(bar_positions, bar_numbers, bar_position_table, bar_number_table,
           W, b):
    pos_i = bar_positions.reshape(N_ROWS // 128, 128).astype(jnp.int32)
    bar_i = bar_numbers.reshape(N_ROWS // 128, 128).astype(jnp.int32)
    table3, idx2 = _prep(bar_position_table, bar_number_table, W,
                         b.reshape(1, D), pos_i, bar_i)
    return _sc_gather(table3.reshape(N_TABLE, D), idx2.reshape(N_ROWS))


# speedup vs baseline: 4.7426x; 1.0070x over previous
import functools

import jax
import jax.numpy as jnp
from jax import lax
from jax.experimental import pallas as pl
from jax.experimental.pallas import tpu as pltpu
from jax.experimental.pallas import tpu_sc as plsc

SEQ = 8192
BATCH = 4
D = 768
HALF = D // 2
MAX_BAR_POS = 96
MAX_BARS = 32
N_ROWS = SEQ * BATCH
N_TABLE = MAX_BAR_POS * MAX_BARS

NC = 2
NS = 16
NW = NC * NS
ROWS_PER_W = N_ROWS // NW
CHUNK = 64
NCHUNK = ROWS_PER_W // CHUNK


def _prep_body(pos_t_ref, num_t_ref, w_ref, b_ref, pi_ref, bi_ref,
               table_ref, idx_ref):
    wp = w_ref[:, :HALF]
    wn = w_ref[:, HALF:]
    p = lax.dot_general(pos_t_ref[:], wp, (((1,), (1,)), ((), ())),
                        preferred_element_type=jnp.float32) + b_ref[:]
    n = lax.dot_general(num_t_ref[:], wn, (((1,), (1,)), ((), ())),
                        preferred_element_type=jnp.float32)
    table_ref[:] = p[:, None, :] + n[None, :, :]
    idx_ref[:] = pi_ref[:] * MAX_BARS + bi_ref[:]


def _prep(pos_t, num_t, w, b2, pos_i, bar_i):
    return pl.pallas_call(
        _prep_body,
        out_shape=(
            jax.ShapeDtypeStruct((MAX_BAR_POS, MAX_BARS, D), jnp.float32),
            jax.ShapeDtypeStruct((N_ROWS // 128, 128), jnp.int32),
        ),
    )(pos_t, num_t, w, b2, pos_i, bar_i)


@functools.partial(
    pl.kernel,
    out_type=jax.ShapeDtypeStruct((SEQ, BATCH, D), jnp.float32),
    mesh=plsc.VectorSubcoreMesh(core_axis_name="c", subcore_axis_name="s"),
    scratch_types=[
        pltpu.VMEM((ROWS_PER_W,), jnp.int32),
        pltpu.VMEM((2, CHUNK, D), jnp.float32),
        pltpu.SemaphoreType.DMA((2,)),
        pltpu.SemaphoreType.DMA((2,)),
    ],
)
def _sc_gather(table_hbm, idx_hbm, out3_hbm, idx_v, rows_v, gsem, wsem):
    out_hbm = out3_hbm.reshape(N_ROWS, D)
    wid = lax.axis_index("s") * NC + lax.axis_index("c")
    base = wid * ROWS_PER_W
    pltpu.sync_copy(idx_hbm.at[pl.ds(base, ROWS_PER_W)], idx_v)
    g_cur = pltpu.async_copy(table_hbm.at[idx_v.at[pl.ds(0, CHUNK)]],
                             rows_v.at[0], gsem.at[0])
    writes = [None, None]
    for c in range(NCHUNK):
        buf = c % 2
        nbuf = (c + 1) % 2
        if c + 1 < NCHUNK:
            if writes[nbuf] is not None:
                writes[nbuf].wait()
            g_next = pltpu.async_copy(
                table_hbm.at[idx_v.at[pl.ds((c + 1) * CHUNK, CHUNK)]],
                rows_v.at[nbuf], gsem.at[nbuf])
        g_cur.wait()
        writes[buf] = pltpu.async_copy(
            rows_v.at[buf], out_hbm.at[pl.ds(base + c * CHUNK, CHUNK)],
            wsem.at[buf])
        if c + 1 < NCHUNK:
            g_cur = g_next
    for w in writes:
        if w is not None:
            w.wait()


def kernel(bar_positions, bar_numbers, bar_position_table, bar_number_table,
           W, b):
    pos_i = bar_positions.reshape(N_ROWS // 128, 128).astype(jnp.int32)
    bar_i = bar_numbers.reshape(N_ROWS // 128, 128).astype(jnp.int32)
    table3, idx2 = _prep(bar_position_table, bar_number_table, W,
                         b.reshape(1, D), pos_i, bar_i)
    return _sc_gather(table3.reshape(N_TABLE, D), idx2.reshape(N_ROWS))
